# Initial kernel scaffold; baseline (speedup 1.0000x reference)
import jax
import jax.numpy as jnp
from jax.experimental import pallas as pl


def kernel(x, lin_tables, W_lin, ffm_tables, W_int, W_att, b_att, H_att, W1, b1, W2, b2, W3, b3, Wo, bo):
    raise NotImplementedError("write your pallas kernel here")


# speedup vs baseline: 1.7190x; 1.7190x over previous
import functools

import jax
import jax.numpy as jnp
import numpy as np
from jax import lax
from jax.experimental import pallas as pl
from jax.experimental.pallas import tpu as pltpu
from jax.experimental.pallas import tpu_sc as plsc

_NF = 26
_VOCAB = 100000
_D = 8
_AF = 4
_B = 4096
_P = _NF * (_NF - 1) // 2

_NC = 2
_NS = 16
_NW = _NC * _NS
_CHUNK = 128
_PER_W = (_NF * _B) // _NW
_N_CH = _PER_W // _CHUNK

_OFF = np.concatenate([[0], np.cumsum([_NF - 1 - r for r in range(_NF - 1)])])

_BT = 256


def _sc_gather(xoff, ffm_flat, lin_flat):
    mesh = plsc.VectorSubcoreMesh(core_axis_name="c", subcore_axis_name="s")

    @functools.partial(
        pl.kernel,
        mesh=mesh,
        compiler_params=pltpu.CompilerParams(use_tc_tiling_on_sc=False),
        out_type=[
            jax.ShapeDtypeStruct((_NF * _B, _D), jnp.float32),
            jax.ShapeDtypeStruct((_NF * _B,), jnp.float32),
        ],
        scratch_types=[
            pltpu.VMEM((_CHUNK,), jnp.int32),
            pltpu.VMEM((_CHUNK, _D), jnp.float32),
            pltpu.VMEM((_CHUNK,), jnp.float32),
            pltpu.SemaphoreType.DMA,
            pltpu.SemaphoreType.DMA,
        ],
    )
    def k(xoff_hbm, ffm_hbm, lin_hbm, emb_out, lin_out, idx_v, rows_v, ln_v,
          sem1, sem2):
        wid = lax.axis_index("s") * _NC + lax.axis_index("c")
        base0 = wid * _PER_W

        def body(j, carry):
            base = pl.multiple_of(base0 + j * _CHUNK, _CHUNK)
            pltpu.sync_copy(xoff_hbm.at[pl.ds(base, _CHUNK)], idx_v)
            cp1 = pltpu.async_copy(ffm_hbm.at[idx_v], rows_v, sem1)
            cp2 = pltpu.async_copy(lin_hbm.at[idx_v], ln_v, sem2)
            cp1.wait()
            cp2.wait()
            pltpu.sync_copy(rows_v, emb_out.at[pl.ds(base, _CHUNK)])
            pltpu.sync_copy(ln_v, lin_out.at[pl.ds(base, _CHUNK)])
            return carry

        lax.fori_loop(0, _N_CH, body, 0)

    return k(xoff, ffm_flat, lin_flat)


def _dense_body(emb_ref, lin_ref, wint_ref, watt_ref, batt_ref, hatt_ref,
                wlin_ref, w1_ref, b1_ref, w2_ref, b2_ref, w3_ref, b3_ref,
                wo_ref, bo_ref, out_ref, bi_ref):
    e = emb_ref[...]
    wint = wint_ref[...]
    G = lax.dot_general(wint, wint, (((1,), (1,)), ((), ())),
                        preferred_element_type=jnp.float32)

    for r in range(_NF - 1):
        kk = _NF - 1 - r
        er = e[:, r:r + 1, :]
        ec = e[:, r + 1:, :]
        g = G[r + 1:, r:r + 1]
        bi_ref[:, _OFF[r]:_OFF[r] + kk, :] = er * ec * g[None, :, :]

    s = None
    for a in range(_AF):
        acc = jnp.full((_P, bi_ref.shape[2]), batt_ref[a], jnp.float32)
        for dd in range(_D):
            acc = acc + bi_ref[dd] * watt_ref[dd, a]
        term = jnp.maximum(acc, 0.0) * hatt_ref[a, 0]
        s = term if s is None else s + term

    m = jnp.max(s, axis=0, keepdims=True)
    ex = jnp.exp(s - m)
    attn = ex / jnp.sum(ex, axis=0, keepdims=True)

    ffm = jnp.concatenate(
        [jnp.sum(attn * bi_ref[dd], axis=0, keepdims=True) for dd in range(_D)],
        axis=0)

    cdim = (((0,), (0,)), ((), ()))
    h = jnp.maximum(lax.dot_general(w1_ref[...], ffm, cdim,
                                    preferred_element_type=jnp.float32)
                    + b1_ref[...], 0.0)
    h = jnp.maximum(lax.dot_general(w2_ref[...], h, cdim,
                                    preferred_element_type=jnp.float32)
                    + b2_ref[...], 0.0)
    h = jnp.maximum(lax.dot_general(w3_ref[...], h, cdim,
                                    preferred_element_type=jnp.float32)
                    + b3_ref[...], 0.0)
    dnn = lax.dot_general(wo_ref[...], h, cdim,
                          preferred_element_type=jnp.float32) + bo_ref[...]
    logits = lax.dot_general(wlin_ref[...], lin_ref[...], cdim,
                             preferred_element_type=jnp.float32)
    out_ref[...] = jax.nn.sigmoid(logits + dnn)


def _dense(emb_t, lin_t, W_int, W_att, b_att, H_att, W_lin,
           W1, b1, W2, b2, W3, b3, Wo, bo, interpret=False):
    grid = (_B // _BT,)
    smem = pl.BlockSpec(memory_space=pltpu.MemorySpace.SMEM)
    return pl.pallas_call(
        _dense_body,
        grid=grid,
        in_specs=[
            pl.BlockSpec((_D, _NF, _BT), lambda i: (0, 0, i)),
            pl.BlockSpec((_NF, _BT), lambda i: (0, i)),
            pl.BlockSpec((_NF, _AF), lambda i: (0, 0)),
            smem,
            smem,
            smem,
            pl.BlockSpec((_NF, 1), lambda i: (0, 0)),
            pl.BlockSpec((_D, 208), lambda i: (0, 0)),
            pl.BlockSpec((208, 1), lambda i: (0, 0)),
            pl.BlockSpec((208, 128), lambda i: (0, 0)),
            pl.BlockSpec((128, 1), lambda i: (0, 0)),
            pl.BlockSpec((128, 64), lambda i: (0, 0)),
            pl.BlockSpec((64, 1), lambda i: (0, 0)),
            pl.BlockSpec((64, 1), lambda i: (0, 0)),
            pl.BlockSpec((1, 1), lambda i: (0, 0)),
        ],
        out_specs=pl.BlockSpec((1, _BT), lambda i: (0, i)),
        out_shape=jax.ShapeDtypeStruct((1, _B), jnp.float32),
        scratch_shapes=[pltpu.VMEM((_D, _P, _BT), jnp.float32)],
        interpret=interpret,
    )(emb_t, lin_t, W_int, W_att, b_att, H_att, W_lin,
      W1, b1, W2, b2, W3, b3, Wo, bo)


def kernel(x, lin_tables, W_lin, ffm_tables, W_int, W_att, b_att, H_att,
           W1, b1, W2, b2, W3, b3, Wo, bo):
    x = x.astype(jnp.int32)
    offs = (jnp.arange(_NF, dtype=jnp.int32) * _VOCAB)[:, None]
    xoff = (x.T + offs).reshape(_NF * _B)
    ffm_flat = ffm_tables.reshape(_NF * _VOCAB, _D)
    lin_flat = lin_tables.reshape(_NF * _VOCAB)

    emb_flat, lin_e = _sc_gather(xoff, ffm_flat, lin_flat)

    emb_t = emb_flat.reshape(_NF, _B, _D).transpose(2, 0, 1)
    lin_t = lin_e.reshape(_NF, _B)

    out = _dense(emb_t, lin_t, W_int, W_att, b_att, H_att, W_lin,
                 W1, b1.reshape(208, 1), W2, b2.reshape(128, 1),
                 W3, b3.reshape(64, 1), Wo, bo.reshape(1, 1))
    return out.reshape(_B, 1)
